# baseline (speedup 1.0000x reference)
import functools

import jax
import jax.numpy as jnp
from jax.experimental import pallas as pl
from jax.experimental.pallas import tpu as pltpu

_DEPTH = 4
_ODEPTH = 4


def _se_ring(x_hbm, w1_ref, w2_ref, o_hbm, ibuf, obuf, in_sems, out_sems,
             *, bt, tiles_per_core):
    core = pl.program_id(0)
    base = core * tiles_per_core

    def start_in(k):
        slot = jax.lax.rem(k, _DEPTH)
        row = (base + k) * bt
        pltpu.make_async_copy(x_hbm.at[pl.ds(row, bt)], ibuf.at[slot],
                              in_sems.at[slot]).start()

    def wait_in(slot):
        pltpu.make_async_copy(ibuf.at[slot], ibuf.at[slot],
                              in_sems.at[slot]).wait()

    def start_out(k):
        slot = jax.lax.rem(k, _ODEPTH)
        row = (base + k) * bt
        pltpu.make_async_copy(obuf.at[slot], o_hbm.at[pl.ds(row, bt)],
                              out_sems.at[slot]).start()

    def wait_out(slot):
        pltpu.make_async_copy(obuf.at[slot], obuf.at[slot],
                              out_sems.at[slot]).wait()

    for k in range(min(_DEPTH, tiles_per_core)):
        start_in(k)

    def body(k, _):
        slot = jax.lax.rem(k, _DEPTH)
        oslot = jax.lax.rem(k, _ODEPTH)
        wait_in(slot)

        @pl.when(k >= _ODEPTH)
        def _():
            wait_out(oslot)

        x = ibuf[slot]
        pooled = jnp.sum(x, axis=2, dtype=jnp.float32)
        hidden = jnp.maximum(
            jnp.dot(pooled, w1_ref[...],
                    preferred_element_type=jnp.float32), 0.0)
        gate = jax.nn.sigmoid(
            jnp.dot(hidden, w2_ref[...],
                    preferred_element_type=jnp.float32))
        obuf[oslot] = x * gate[:, :, None]
        start_out(k)

        @pl.when(k + _DEPTH < tiles_per_core)
        def _():
            start_in(k + _DEPTH)
        return ()

    jax.lax.fori_loop(0, tiles_per_core, body, ())
    for j in range(max(0, tiles_per_core - _ODEPTH), tiles_per_core):
        wait_out(jax.lax.rem(j, _ODEPTH))


@functools.partial(jax.jit, static_argnames=("bt",))
def _se_apply(x, w1, w2, bt=2):
    B, C, H, W = x.shape
    HW = H * W
    Cr = w1.shape[1]

    x3 = x.reshape(B, C, HW)
    w1_pre = w1.astype(jnp.float32) * jnp.float32(1.0 / HW)
    w2_f = w2.astype(jnp.float32)

    n_cores = 2 if (B // bt) % 2 == 0 else 1
    tiles_per_core = B // bt // n_cores

    out = pl.pallas_call(
        functools.partial(_se_ring, bt=bt, tiles_per_core=tiles_per_core),
        out_shape=jax.ShapeDtypeStruct((B, C, HW), x.dtype),
        grid=(n_cores,),
        in_specs=[
            pl.BlockSpec(memory_space=pl.ANY),
            pl.BlockSpec((C, Cr), lambda i: (0, 0)),
            pl.BlockSpec((Cr, C), lambda i: (0, 0)),
        ],
        out_specs=pl.BlockSpec(memory_space=pl.ANY),
        scratch_shapes=[
            pltpu.VMEM((_DEPTH, bt, C, HW), jnp.float32),
            pltpu.VMEM((_ODEPTH, bt, C, HW), jnp.float32),
            pltpu.SemaphoreType.DMA((_DEPTH,)),
            pltpu.SemaphoreType.DMA((_ODEPTH,)),
        ],
        compiler_params=pltpu.CompilerParams(
            dimension_semantics=("parallel",),
            vmem_limit_bytes=60 * 1024 * 1024,
        ),
    )(x3, w1_pre, w2_f)
    return out.reshape(B, C, H, W)


def kernel(x, w1, w2):
    return _se_apply(x, w1, w2)
